# baseline (device time: 47175 ns/iter reference)
import jax
import jax.numpy as jnp
from jax import lax
from jax.experimental import pallas as pl
from jax.experimental.pallas import tpu as pltpu

_K = 16
_NR = 8
_SUB = 8
_NSLOT = 7 * _SUB // 2

_G = [s for pair in zip(range(_SUB // 2), range(_SUB // 2, _SUB)) for s in pair]


def _ring_coords(p):
    y = (p >= _NR // 2).astype(jnp.int32)
    z = jnp.where(p < _NR // 2, p, _NR - 1 - p)
    return y, z


def kernel(ids, E):
    v_local, d = E.shape
    t = ids.shape[0]
    c = t // _NR
    sz = c // _SUB

    my_x = lax.axis_index("x")
    my_y = lax.axis_index("y")
    my_z = lax.axis_index("z")
    r = jnp.where(my_y == 0, my_z, _NR - 1 - my_z).astype(jnp.int32)

    lids = ids - my_x * v_local
    cl_all = jnp.clip(lids, 0, v_local - 1).astype(jnp.int32)
    mask_all = ((lids >= 0) & (lids < v_local)).astype(jnp.float32)[:, None]
    cl = lax.dynamic_slice(cl_all, (r * c,), (c,))
    maskf = lax.dynamic_slice(mask_all, (r * c, 0), (c, 1))

    def body(cl_ref, mask_ref, e_ref, out_ref, part_ref, pbf_ref,
             comm_ref, obf_ref, gsems, xs_sems, xr_sems,
             cw_s, cw_r, ccw_s, ccw_r):
        x = lax.axis_index("x")
        y = lax.axis_index("y")
        z = lax.axis_index("z")
        rr = jnp.where(y == 0, z, _NR - 1 - z).astype(jnp.int32)
        xpeer = (1 - x, y, z)
        ry, rz = _ring_coords(lax.rem(rr + 1, _NR))
        ly, lz = _ring_coords(lax.rem(rr + _NR - 1, _NR))
        right = (x, ry, rz)
        left = (x, ly, lz)

        barrier = pltpu.get_barrier_semaphore()
        for nbr in (xpeer, left, right):
            pl.semaphore_signal(
                barrier, inc=1, device_id=nbr,
                device_id_type=pl.DeviceIdType.MESH,
            )
        pl.semaphore_wait(barrier, 3)

        def exch_d(s):
            return pltpu.make_async_remote_copy(
                src_ref=pbf_ref.at[pl.ds(s * sz, sz), :],
                dst_ref=comm_ref.at[pl.ds(s * sz, sz), :],
                send_sem=xs_sems.at[s], recv_sem=xr_sems.at[s],
                device_id=xpeer, device_id_type=pl.DeviceIdType.MESH,
            )

        def sub_slice(chunk_idx, sub):
            return obf_ref.at[pl.ds(chunk_idx * c + sub * sz, sz), :]

        def cw_sub(m):
            return m % _SUB

        def ccw_sub(m):
            return (_SUB // 2 + m) % _SUB

        def cw_send_d(m):
            ref = sub_slice(lax.rem(rr - m // _SUB + _NR, _NR), cw_sub(m))
            return pltpu.make_async_remote_copy(
                src_ref=ref, dst_ref=ref,
                send_sem=cw_s.at[m], recv_sem=cw_r.at[m],
                device_id=right, device_id_type=pl.DeviceIdType.MESH,
            )

        def cw_recv_d(m):
            ref = sub_slice(lax.rem(rr - 1 - m // _SUB + _NR, _NR), cw_sub(m))
            return pltpu.make_async_remote_copy(
                src_ref=ref, dst_ref=ref,
                send_sem=cw_s.at[m], recv_sem=cw_r.at[m],
                device_id=left, device_id_type=pl.DeviceIdType.MESH,
            )

        def ccw_send_d(m):
            ref = sub_slice(lax.rem(rr + m // _SUB, _NR), ccw_sub(m))
            return pltpu.make_async_remote_copy(
                src_ref=ref, dst_ref=ref,
                send_sem=ccw_s.at[m], recv_sem=ccw_r.at[m],
                device_id=left, device_id_type=pl.DeviceIdType.MESH,
            )

        def ccw_recv_d(m):
            ref = sub_slice(lax.rem(rr + 1 + m // _SUB, _NR), ccw_sub(m))
            return pltpu.make_async_remote_copy(
                src_ref=ref, dst_ref=ref,
                send_sem=ccw_s.at[m], recv_sem=ccw_r.at[m],
                device_id=right, device_id_type=pl.DeviceIdType.MESH,
            )

        def gather_sub(s):
            def row_copy(i):
                return pltpu.make_async_copy(
                    e_ref.at[pl.ds(cl_ref[i], 1), :],
                    part_ref.at[pl.ds(i, 1), :],
                    gsems.at[lax.rem(i, _K)],
                )

            lo = s * sz

            def gather_step(i, carry):
                @pl.when(i >= lo + _K)
                def _():
                    row_copy(i - _K).wait()
                row_copy(i).start()
                return carry

            lax.fori_loop(lo, lo + sz, gather_step, 0)

            def drain_step(j, carry):
                row_copy(lo + sz - _K + j).wait()
                return carry

            lax.fori_loop(0, _K, drain_step, 0)
            pbf_ref[pl.ds(lo, sz), :] = (
                part_ref[pl.ds(lo, sz), :] * mask_ref[pl.ds(lo, sz), :]
            ).astype(jnp.bfloat16)

        def process_sub(s):
            exch_d(s).wait_recv()
            obf_ref[pl.ds(rr * c + s * sz, sz), :] = (
                pbf_ref[pl.ds(s * sz, sz), :]
                + comm_ref[pl.ds(s * sz, sz), :]
            )
            cw_send_d(s).start()
            ccw_send_d((s - _SUB // 2) % _SUB).start()

        for i, s in enumerate(_G):
            gather_sub(s)
            exch_d(s).start()
            if i > 0:
                process_sub(_G[i - 1])
        process_sub(_G[-1])

        for m in range(_SUB, _NSLOT):
            cw_recv_d(m - _SUB).wait_recv()
            cw_send_d(m).start()
            ccw_recv_d(m - _SUB).wait_recv()
            ccw_send_d(m).start()

        for m in range(_NSLOT - _SUB, _NSLOT):
            cw_recv_d(m).wait_recv()
            ccw_recv_d(m).wait_recv()

        out_ref[...] = obf_ref[...].astype(jnp.float32)

        for s in range(_SUB):
            exch_d(s).wait_send()
        for m in range(_NSLOT):
            cw_send_d(m).wait_send()
            ccw_send_d(m).wait_send()

    return pl.pallas_call(
        body,
        out_shape=jax.ShapeDtypeStruct((t, d), jnp.float32),
        in_specs=[
            pl.BlockSpec(memory_space=pltpu.SMEM),
            pl.BlockSpec(memory_space=pltpu.VMEM),
            pl.BlockSpec(memory_space=pl.ANY),
        ],
        out_specs=pl.BlockSpec(memory_space=pltpu.VMEM),
        scratch_shapes=[
            pltpu.VMEM((c, d), jnp.float32),
            pltpu.VMEM((c, d), jnp.bfloat16),
            pltpu.VMEM((c, d), jnp.bfloat16),
            pltpu.VMEM((t, d), jnp.bfloat16),
            pltpu.SemaphoreType.DMA((_K,)),
            pltpu.SemaphoreType.DMA((_SUB,)),
            pltpu.SemaphoreType.DMA((_SUB,)),
            pltpu.SemaphoreType.DMA((_NSLOT,)),
            pltpu.SemaphoreType.DMA((_NSLOT,)),
            pltpu.SemaphoreType.DMA((_NSLOT,)),
            pltpu.SemaphoreType.DMA((_NSLOT,)),
        ],
        compiler_params=pltpu.CompilerParams(collective_id=0),
    )(cl, maskf, E)


# device time: 45417 ns/iter; 1.0387x vs baseline; 1.0387x over previous
import jax
import jax.numpy as jnp
from jax import lax
from jax.experimental import pallas as pl
from jax.experimental.pallas import tpu as pltpu

_K = 16
_NR = 8
_SUB = 4
_NSLOT = 7 * _SUB // 2

_G = [s for pair in zip(range(_SUB // 2), range(_SUB // 2, _SUB)) for s in pair]


def _ring_coords(p):
    y = (p >= _NR // 2).astype(jnp.int32)
    z = jnp.where(p < _NR // 2, p, _NR - 1 - p)
    return y, z


def kernel(ids, E):
    v_local, d = E.shape
    t = ids.shape[0]
    c = t // _NR
    sz = c // _SUB

    my_x = lax.axis_index("x")
    my_y = lax.axis_index("y")
    my_z = lax.axis_index("z")
    r = jnp.where(my_y == 0, my_z, _NR - 1 - my_z).astype(jnp.int32)

    lids = ids - my_x * v_local
    cl_all = jnp.clip(lids, 0, v_local - 1).astype(jnp.int32)
    mask_all = ((lids >= 0) & (lids < v_local)).astype(jnp.float32)[:, None]
    cl = lax.dynamic_slice(cl_all, (r * c,), (c,))
    maskf = lax.dynamic_slice(mask_all, (r * c, 0), (c, 1))

    def body(cl_ref, mask_ref, e_ref, out_ref, part_ref, pbf_ref,
             comm_ref, obf_ref, gsems, xs_sems, xr_sems,
             cw_s, cw_r, ccw_s, ccw_r):
        x = lax.axis_index("x")
        y = lax.axis_index("y")
        z = lax.axis_index("z")
        rr = jnp.where(y == 0, z, _NR - 1 - z).astype(jnp.int32)
        xpeer = (1 - x, y, z)
        ry, rz = _ring_coords(lax.rem(rr + 1, _NR))
        ly, lz = _ring_coords(lax.rem(rr + _NR - 1, _NR))
        right = (x, ry, rz)
        left = (x, ly, lz)

        barrier = pltpu.get_barrier_semaphore()
        for nbr in (xpeer, left, right):
            pl.semaphore_signal(
                barrier, inc=1, device_id=nbr,
                device_id_type=pl.DeviceIdType.MESH,
            )
        pl.semaphore_wait(barrier, 3)

        def exch_d(s):
            return pltpu.make_async_remote_copy(
                src_ref=pbf_ref.at[pl.ds(s * sz, sz), :],
                dst_ref=comm_ref.at[pl.ds(s * sz, sz), :],
                send_sem=xs_sems.at[s], recv_sem=xr_sems.at[s],
                device_id=xpeer, device_id_type=pl.DeviceIdType.MESH,
            )

        def sub_slice(chunk_idx, sub):
            return obf_ref.at[pl.ds(chunk_idx * c + sub * sz, sz), :]

        def cw_sub(m):
            return m % _SUB

        def ccw_sub(m):
            return (_SUB // 2 + m) % _SUB

        def cw_send_d(m):
            ref = sub_slice(lax.rem(rr - m // _SUB + _NR, _NR), cw_sub(m))
            return pltpu.make_async_remote_copy(
                src_ref=ref, dst_ref=ref,
                send_sem=cw_s.at[m], recv_sem=cw_r.at[m],
                device_id=right, device_id_type=pl.DeviceIdType.MESH,
            )

        def cw_recv_d(m):
            ref = sub_slice(lax.rem(rr - 1 - m // _SUB + _NR, _NR), cw_sub(m))
            return pltpu.make_async_remote_copy(
                src_ref=ref, dst_ref=ref,
                send_sem=cw_s.at[m], recv_sem=cw_r.at[m],
                device_id=left, device_id_type=pl.DeviceIdType.MESH,
            )

        def ccw_send_d(m):
            ref = sub_slice(lax.rem(rr + m // _SUB, _NR), ccw_sub(m))
            return pltpu.make_async_remote_copy(
                src_ref=ref, dst_ref=ref,
                send_sem=ccw_s.at[m], recv_sem=ccw_r.at[m],
                device_id=left, device_id_type=pl.DeviceIdType.MESH,
            )

        def ccw_recv_d(m):
            ref = sub_slice(lax.rem(rr + 1 + m // _SUB, _NR), ccw_sub(m))
            return pltpu.make_async_remote_copy(
                src_ref=ref, dst_ref=ref,
                send_sem=ccw_s.at[m], recv_sem=ccw_r.at[m],
                device_id=right, device_id_type=pl.DeviceIdType.MESH,
            )

        def gather_sub(s):
            def row_copy(i):
                return pltpu.make_async_copy(
                    e_ref.at[pl.ds(cl_ref[i], 1), :],
                    part_ref.at[pl.ds(i, 1), :],
                    gsems.at[lax.rem(i, _K)],
                )

            lo = s * sz

            def gather_step(i, carry):
                @pl.when(i >= lo + _K)
                def _():
                    row_copy(i - _K).wait()
                row_copy(i).start()
                return carry

            lax.fori_loop(lo, lo + sz, gather_step, 0)

            def drain_step(j, carry):
                row_copy(lo + sz - _K + j).wait()
                return carry

            lax.fori_loop(0, _K, drain_step, 0)
            pbf_ref[pl.ds(lo, sz), :] = (
                part_ref[pl.ds(lo, sz), :] * mask_ref[pl.ds(lo, sz), :]
            ).astype(jnp.bfloat16)

        def dequant(chunk_idx, sub):
            lo = chunk_idx * c + sub * sz
            out_ref[pl.ds(lo, sz), :] = (
                obf_ref[pl.ds(lo, sz), :].astype(jnp.float32)
            )

        def dequant_cw(m):
            dequant(lax.rem(rr - 1 - m // _SUB + _NR, _NR), cw_sub(m))

        def dequant_ccw(m):
            dequant(lax.rem(rr + 1 + m // _SUB, _NR), ccw_sub(m))

        def process_sub(s):
            exch_d(s).wait_recv()
            obf_ref[pl.ds(rr * c + s * sz, sz), :] = (
                pbf_ref[pl.ds(s * sz, sz), :]
                + comm_ref[pl.ds(s * sz, sz), :]
            )
            cw_send_d(s).start()
            ccw_send_d((s - _SUB // 2) % _SUB).start()
            dequant(rr, s)

        for i, s in enumerate(_G):
            gather_sub(s)
            exch_d(s).start()
            if i > 0:
                process_sub(_G[i - 1])
        process_sub(_G[-1])

        for m in range(_SUB, _NSLOT):
            cw_recv_d(m - _SUB).wait_recv()
            cw_send_d(m).start()
            ccw_recv_d(m - _SUB).wait_recv()
            ccw_send_d(m).start()
            dequant_cw(m - _SUB)
            dequant_ccw(m - _SUB)

        for m in range(_NSLOT - _SUB, _NSLOT):
            cw_recv_d(m).wait_recv()
            dequant_cw(m)
            ccw_recv_d(m).wait_recv()
            dequant_ccw(m)

        for s in range(_SUB):
            exch_d(s).wait_send()
        for m in range(_NSLOT):
            cw_send_d(m).wait_send()
            ccw_send_d(m).wait_send()

    return pl.pallas_call(
        body,
        out_shape=jax.ShapeDtypeStruct((t, d), jnp.float32),
        in_specs=[
            pl.BlockSpec(memory_space=pltpu.SMEM),
            pl.BlockSpec(memory_space=pltpu.VMEM),
            pl.BlockSpec(memory_space=pl.ANY),
        ],
        out_specs=pl.BlockSpec(memory_space=pltpu.VMEM),
        scratch_shapes=[
            pltpu.VMEM((c, d), jnp.float32),
            pltpu.VMEM((c, d), jnp.bfloat16),
            pltpu.VMEM((c, d), jnp.bfloat16),
            pltpu.VMEM((t, d), jnp.bfloat16),
            pltpu.SemaphoreType.DMA((_K,)),
            pltpu.SemaphoreType.DMA((_SUB,)),
            pltpu.SemaphoreType.DMA((_SUB,)),
            pltpu.SemaphoreType.DMA((_NSLOT,)),
            pltpu.SemaphoreType.DMA((_NSLOT,)),
            pltpu.SemaphoreType.DMA((_NSLOT,)),
            pltpu.SemaphoreType.DMA((_NSLOT,)),
        ],
        compiler_params=pltpu.CompilerParams(collective_id=0),
    )(cl, maskf, E)


# device time: 40744 ns/iter; 1.1578x vs baseline; 1.1147x over previous
import jax
import jax.numpy as jnp
from jax import lax
from jax.experimental import pallas as pl
from jax.experimental.pallas import tpu as pltpu

_K = 32
_NR = 8
_SUB = 4
_NSLOT = 7 * _SUB // 2

_G = [s for pair in zip(range(_SUB // 2), range(_SUB // 2, _SUB)) for s in pair]


def _ring_coords(p):
    y = (p >= _NR // 2).astype(jnp.int32)
    z = jnp.where(p < _NR // 2, p, _NR - 1 - p)
    return y, z


def kernel(ids, E):
    v_local, d = E.shape
    t = ids.shape[0]
    c = t // _NR
    sz = c // _SUB

    my_x = lax.axis_index("x")
    my_y = lax.axis_index("y")
    my_z = lax.axis_index("z")
    r = jnp.where(my_y == 0, my_z, _NR - 1 - my_z).astype(jnp.int32)

    lids = ids - my_x * v_local
    cl_all = jnp.clip(lids, 0, v_local - 1).astype(jnp.int32)
    mask_all = ((lids >= 0) & (lids < v_local)).astype(jnp.float32)[:, None]
    cl = lax.dynamic_slice(cl_all, (r * c,), (c,))
    maskf = lax.dynamic_slice(mask_all, (r * c, 0), (c, 1))

    def body(cl_ref, mask_ref, e_ref, out_ref, part_ref, pbf_ref,
             comm_ref, obf_ref, gsems, xs_sems, xr_sems,
             cw_s, cw_r, ccw_s, ccw_r):
        x = lax.axis_index("x")
        y = lax.axis_index("y")
        z = lax.axis_index("z")
        rr = jnp.where(y == 0, z, _NR - 1 - z).astype(jnp.int32)
        xpeer = (1 - x, y, z)
        ry, rz = _ring_coords(lax.rem(rr + 1, _NR))
        ly, lz = _ring_coords(lax.rem(rr + _NR - 1, _NR))
        right = (x, ry, rz)
        left = (x, ly, lz)

        barrier = pltpu.get_barrier_semaphore()
        for nbr in (xpeer, left, right):
            pl.semaphore_signal(
                barrier, inc=1, device_id=nbr,
                device_id_type=pl.DeviceIdType.MESH,
            )
        pl.semaphore_wait(barrier, 3)

        def exch_d(s):
            return pltpu.make_async_remote_copy(
                src_ref=pbf_ref.at[pl.ds(s * sz, sz), :],
                dst_ref=comm_ref.at[pl.ds(s * sz, sz), :],
                send_sem=xs_sems.at[s], recv_sem=xr_sems.at[s],
                device_id=xpeer, device_id_type=pl.DeviceIdType.MESH,
            )

        def sub_slice(chunk_idx, sub):
            return obf_ref.at[pl.ds(chunk_idx * c + sub * sz, sz), :]

        def cw_sub(m):
            return m % _SUB

        def ccw_sub(m):
            return (_SUB // 2 + m) % _SUB

        def cw_send_d(m):
            ref = sub_slice(lax.rem(rr - m // _SUB + _NR, _NR), cw_sub(m))
            return pltpu.make_async_remote_copy(
                src_ref=ref, dst_ref=ref,
                send_sem=cw_s.at[m], recv_sem=cw_r.at[m],
                device_id=right, device_id_type=pl.DeviceIdType.MESH,
            )

        def cw_recv_d(m):
            ref = sub_slice(lax.rem(rr - 1 - m // _SUB + _NR, _NR), cw_sub(m))
            return pltpu.make_async_remote_copy(
                src_ref=ref, dst_ref=ref,
                send_sem=cw_s.at[m], recv_sem=cw_r.at[m],
                device_id=left, device_id_type=pl.DeviceIdType.MESH,
            )

        def ccw_send_d(m):
            ref = sub_slice(lax.rem(rr + m // _SUB, _NR), ccw_sub(m))
            return pltpu.make_async_remote_copy(
                src_ref=ref, dst_ref=ref,
                send_sem=ccw_s.at[m], recv_sem=ccw_r.at[m],
                device_id=left, device_id_type=pl.DeviceIdType.MESH,
            )

        def ccw_recv_d(m):
            ref = sub_slice(lax.rem(rr + 1 + m // _SUB, _NR), ccw_sub(m))
            return pltpu.make_async_remote_copy(
                src_ref=ref, dst_ref=ref,
                send_sem=ccw_s.at[m], recv_sem=ccw_r.at[m],
                device_id=right, device_id_type=pl.DeviceIdType.MESH,
            )

        def gather_sub(s):
            def row_copy(i):
                return pltpu.make_async_copy(
                    e_ref.at[pl.ds(cl_ref[i], 1), :],
                    part_ref.at[pl.ds(i, 1), :],
                    gsems.at[lax.rem(i, _K)],
                )

            lo = s * sz

            def gather_step(i, carry):
                @pl.when(i >= lo + _K)
                def _():
                    row_copy(i - _K).wait()
                row_copy(i).start()
                return carry

            lax.fori_loop(lo, lo + sz, gather_step, 0)

            def drain_step(j, carry):
                row_copy(lo + sz - _K + j).wait()
                return carry

            lax.fori_loop(0, _K, drain_step, 0)
            pbf_ref[pl.ds(lo, sz), :] = (
                part_ref[pl.ds(lo, sz), :] * mask_ref[pl.ds(lo, sz), :]
            ).astype(jnp.bfloat16)

        def dequant(chunk_idx, sub):
            lo = chunk_idx * c + sub * sz
            out_ref[pl.ds(lo, sz), :] = (
                obf_ref[pl.ds(lo, sz), :].astype(jnp.float32)
            )

        def dequant_cw(m):
            dequant(lax.rem(rr - 1 - m // _SUB + _NR, _NR), cw_sub(m))

        def dequant_ccw(m):
            dequant(lax.rem(rr + 1 + m // _SUB, _NR), ccw_sub(m))

        def process_sub(s):
            exch_d(s).wait_recv()
            obf_ref[pl.ds(rr * c + s * sz, sz), :] = (
                pbf_ref[pl.ds(s * sz, sz), :]
                + comm_ref[pl.ds(s * sz, sz), :]
            )
            cw_send_d(s).start()
            ccw_send_d((s - _SUB // 2) % _SUB).start()
            dequant(rr, s)

        for i, s in enumerate(_G):
            gather_sub(s)
            exch_d(s).start()
            if i > 0:
                process_sub(_G[i - 1])
        process_sub(_G[-1])

        for m in range(_SUB, _NSLOT):
            cw_recv_d(m - _SUB).wait_recv()
            cw_send_d(m).start()
            ccw_recv_d(m - _SUB).wait_recv()
            ccw_send_d(m).start()
            dequant_cw(m - _SUB)
            dequant_ccw(m - _SUB)

        for m in range(_NSLOT - _SUB, _NSLOT):
            cw_recv_d(m).wait_recv()
            dequant_cw(m)
            ccw_recv_d(m).wait_recv()
            dequant_ccw(m)

        for s in range(_SUB):
            exch_d(s).wait_send()
        for m in range(_NSLOT):
            cw_send_d(m).wait_send()
            ccw_send_d(m).wait_send()

    return pl.pallas_call(
        body,
        out_shape=jax.ShapeDtypeStruct((t, d), jnp.float32),
        in_specs=[
            pl.BlockSpec(memory_space=pltpu.SMEM),
            pl.BlockSpec(memory_space=pltpu.VMEM),
            pl.BlockSpec(memory_space=pl.ANY),
        ],
        out_specs=pl.BlockSpec(memory_space=pltpu.VMEM),
        scratch_shapes=[
            pltpu.VMEM((c, d), jnp.float32),
            pltpu.VMEM((c, d), jnp.bfloat16),
            pltpu.VMEM((c, d), jnp.bfloat16),
            pltpu.VMEM((t, d), jnp.bfloat16),
            pltpu.SemaphoreType.DMA((_K,)),
            pltpu.SemaphoreType.DMA((_SUB,)),
            pltpu.SemaphoreType.DMA((_SUB,)),
            pltpu.SemaphoreType.DMA((_NSLOT,)),
            pltpu.SemaphoreType.DMA((_NSLOT,)),
            pltpu.SemaphoreType.DMA((_NSLOT,)),
            pltpu.SemaphoreType.DMA((_NSLOT,)),
        ],
        compiler_params=pltpu.CompilerParams(collective_id=0),
    )(cl, maskf, E)


# device time: 40418 ns/iter; 1.1672x vs baseline; 1.0081x over previous
import jax
import jax.numpy as jnp
from jax import lax
from jax.experimental import pallas as pl
from jax.experimental.pallas import tpu as pltpu

_K = 64
_NR = 8
_SUB = 4
_NSLOT = 7 * _SUB // 2

_G = [s for pair in zip(range(_SUB // 2), range(_SUB // 2, _SUB)) for s in pair]


def _ring_coords(p):
    y = (p >= _NR // 2).astype(jnp.int32)
    z = jnp.where(p < _NR // 2, p, _NR - 1 - p)
    return y, z


def kernel(ids, E):
    v_local, d = E.shape
    t = ids.shape[0]
    c = t // _NR
    sz = c // _SUB

    my_x = lax.axis_index("x")
    my_y = lax.axis_index("y")
    my_z = lax.axis_index("z")
    r = jnp.where(my_y == 0, my_z, _NR - 1 - my_z).astype(jnp.int32)

    lids = ids - my_x * v_local
    cl_all = jnp.clip(lids, 0, v_local - 1).astype(jnp.int32)
    mask_all = ((lids >= 0) & (lids < v_local)).astype(jnp.float32)[:, None]
    cl = lax.dynamic_slice(cl_all, (r * c,), (c,))
    maskf = lax.dynamic_slice(mask_all, (r * c, 0), (c, 1))

    def body(cl_ref, mask_ref, e_ref, out_ref, part_ref, pbf_ref,
             comm_ref, obf_ref, gsems, xs_sems, xr_sems,
             cw_s, cw_r, ccw_s, ccw_r):
        x = lax.axis_index("x")
        y = lax.axis_index("y")
        z = lax.axis_index("z")
        rr = jnp.where(y == 0, z, _NR - 1 - z).astype(jnp.int32)
        xpeer = (1 - x, y, z)
        ry, rz = _ring_coords(lax.rem(rr + 1, _NR))
        ly, lz = _ring_coords(lax.rem(rr + _NR - 1, _NR))
        right = (x, ry, rz)
        left = (x, ly, lz)

        barrier = pltpu.get_barrier_semaphore()
        for nbr in (xpeer, left, right):
            pl.semaphore_signal(
                barrier, inc=1, device_id=nbr,
                device_id_type=pl.DeviceIdType.MESH,
            )
        pl.semaphore_wait(barrier, 3)

        def exch_d(s):
            return pltpu.make_async_remote_copy(
                src_ref=pbf_ref.at[pl.ds(s * sz, sz), :],
                dst_ref=comm_ref.at[pl.ds(s * sz, sz), :],
                send_sem=xs_sems.at[s], recv_sem=xr_sems.at[s],
                device_id=xpeer, device_id_type=pl.DeviceIdType.MESH,
            )

        def sub_slice(chunk_idx, sub):
            return obf_ref.at[pl.ds(chunk_idx * c + sub * sz, sz), :]

        def cw_sub(m):
            return m % _SUB

        def ccw_sub(m):
            return (_SUB // 2 + m) % _SUB

        def cw_send_d(m):
            ref = sub_slice(lax.rem(rr - m // _SUB + _NR, _NR), cw_sub(m))
            return pltpu.make_async_remote_copy(
                src_ref=ref, dst_ref=ref,
                send_sem=cw_s.at[m], recv_sem=cw_r.at[m],
                device_id=right, device_id_type=pl.DeviceIdType.MESH,
            )

        def cw_recv_d(m):
            ref = sub_slice(lax.rem(rr - 1 - m // _SUB + _NR, _NR), cw_sub(m))
            return pltpu.make_async_remote_copy(
                src_ref=ref, dst_ref=ref,
                send_sem=cw_s.at[m], recv_sem=cw_r.at[m],
                device_id=left, device_id_type=pl.DeviceIdType.MESH,
            )

        def ccw_send_d(m):
            ref = sub_slice(lax.rem(rr + m // _SUB, _NR), ccw_sub(m))
            return pltpu.make_async_remote_copy(
                src_ref=ref, dst_ref=ref,
                send_sem=ccw_s.at[m], recv_sem=ccw_r.at[m],
                device_id=left, device_id_type=pl.DeviceIdType.MESH,
            )

        def ccw_recv_d(m):
            ref = sub_slice(lax.rem(rr + 1 + m // _SUB, _NR), ccw_sub(m))
            return pltpu.make_async_remote_copy(
                src_ref=ref, dst_ref=ref,
                send_sem=ccw_s.at[m], recv_sem=ccw_r.at[m],
                device_id=right, device_id_type=pl.DeviceIdType.MESH,
            )

        def gather_sub(s):
            def row_copy(i):
                return pltpu.make_async_copy(
                    e_ref.at[pl.ds(cl_ref[i], 1), :],
                    part_ref.at[pl.ds(i, 1), :],
                    gsems.at[lax.rem(i, _K)],
                )

            lo = s * sz

            def gather_step(i, carry):
                @pl.when(i >= lo + _K)
                def _():
                    row_copy(i - _K).wait()
                row_copy(i).start()
                return carry

            lax.fori_loop(lo, lo + sz, gather_step, 0)

            def drain_step(j, carry):
                row_copy(lo + sz - _K + j).wait()
                return carry

            lax.fori_loop(0, _K, drain_step, 0)
            pbf_ref[pl.ds(lo, sz), :] = (
                part_ref[pl.ds(lo, sz), :] * mask_ref[pl.ds(lo, sz), :]
            ).astype(jnp.bfloat16)

        def dequant(chunk_idx, sub):
            lo = chunk_idx * c + sub * sz
            out_ref[pl.ds(lo, sz), :] = (
                obf_ref[pl.ds(lo, sz), :].astype(jnp.float32)
            )

        def dequant_cw(m):
            dequant(lax.rem(rr - 1 - m // _SUB + _NR, _NR), cw_sub(m))

        def dequant_ccw(m):
            dequant(lax.rem(rr + 1 + m // _SUB, _NR), ccw_sub(m))

        def process_sub(s):
            exch_d(s).wait_recv()
            obf_ref[pl.ds(rr * c + s * sz, sz), :] = (
                pbf_ref[pl.ds(s * sz, sz), :]
                + comm_ref[pl.ds(s * sz, sz), :]
            )
            cw_send_d(s).start()
            ccw_send_d((s - _SUB // 2) % _SUB).start()
            dequant(rr, s)

        for i, s in enumerate(_G):
            gather_sub(s)
            exch_d(s).start()
            if i > 0:
                process_sub(_G[i - 1])
        process_sub(_G[-1])

        for m in range(_SUB, _NSLOT):
            cw_recv_d(m - _SUB).wait_recv()
            cw_send_d(m).start()
            ccw_recv_d(m - _SUB).wait_recv()
            ccw_send_d(m).start()
            dequant_cw(m - _SUB)
            dequant_ccw(m - _SUB)

        for m in range(_NSLOT - _SUB, _NSLOT):
            cw_recv_d(m).wait_recv()
            dequant_cw(m)
            ccw_recv_d(m).wait_recv()
            dequant_ccw(m)

        for s in range(_SUB):
            exch_d(s).wait_send()
        for m in range(_NSLOT):
            cw_send_d(m).wait_send()
            ccw_send_d(m).wait_send()

    return pl.pallas_call(
        body,
        out_shape=jax.ShapeDtypeStruct((t, d), jnp.float32),
        in_specs=[
            pl.BlockSpec(memory_space=pltpu.SMEM),
            pl.BlockSpec(memory_space=pltpu.VMEM),
            pl.BlockSpec(memory_space=pl.ANY),
        ],
        out_specs=pl.BlockSpec(memory_space=pltpu.VMEM),
        scratch_shapes=[
            pltpu.VMEM((c, d), jnp.float32),
            pltpu.VMEM((c, d), jnp.bfloat16),
            pltpu.VMEM((c, d), jnp.bfloat16),
            pltpu.VMEM((t, d), jnp.bfloat16),
            pltpu.SemaphoreType.DMA((_K,)),
            pltpu.SemaphoreType.DMA((_SUB,)),
            pltpu.SemaphoreType.DMA((_SUB,)),
            pltpu.SemaphoreType.DMA((_NSLOT,)),
            pltpu.SemaphoreType.DMA((_NSLOT,)),
            pltpu.SemaphoreType.DMA((_NSLOT,)),
            pltpu.SemaphoreType.DMA((_NSLOT,)),
        ],
        compiler_params=pltpu.CompilerParams(collective_id=0),
    )(cl, maskf, E)
